# baseline (speedup 1.0000x reference)
import functools

import jax
import jax.numpy as jnp
from jax import lax
from jax.experimental import pallas as pl
from jax.experimental.pallas import tpu as pltpu
from jax.experimental.pallas import tpu_sc as plsc


def kernel(tokenized_prompts, ctx, token_embedding):
    n_cls, seq = tokenized_prompts.shape
    n_ctx, d = ctx.shape
    vocab = token_embedding.shape[0]
    seq_p = 80

    table = jnp.concatenate([token_embedding, ctx], axis=0)
    pos = jnp.arange(seq, dtype=jnp.int32)[None, :]
    ctx_ids = (vocab - 1 + pos).astype(jnp.int32)
    idx = jnp.where((pos >= 1) & (pos < 1 + n_ctx), ctx_ids,
                    tokenized_prompts)
    idx = jnp.pad(idx, ((0, 0), (0, 128 - seq)))

    info = plsc.get_sparse_core_info()
    nc, ns = info.num_cores, info.num_subcores
    nw = nc * ns
    per_w = n_cls // nw
    nbuf = 3
    chunk = 32
    n_ch = per_w // chunk

    mesh = plsc.VectorSubcoreMesh(core_axis_name="c", subcore_axis_name="s")

    @functools.partial(
        pl.kernel,
        out_type=jax.ShapeDtypeStruct((n_cls, seq, d), jnp.float32),
        mesh=mesh,
        scratch_types=[
            pltpu.VMEM((chunk, 128), jnp.int32),
            pltpu.VMEM((nbuf, seq_p, d), jnp.float32),
            [pltpu.SemaphoreType.DMA] * nbuf,
            [pltpu.SemaphoreType.DMA] * nbuf,
        ],
    )
    def _gather_kernel(idx_hbm, table_hbm, out_hbm, idx_v, rows_v,
                       gsems, ssems):
        wid = lax.axis_index("s") * nc + lax.axis_index("c")
        base = wid * per_w

        @pl.loop(0, n_ch)
        def _outer(j):
            cbase = base + j * chunk
            pltpu.sync_copy(idx_hbm.at[pl.ds(cbase, chunk)], idx_v)

            def gather_desc(k, b):
                return pltpu.make_async_copy(
                    table_hbm.at[idx_v.at[k, pl.ds(0, seq_p)]],
                    rows_v.at[b], gsems[b])

            def store_desc(k, b):
                row = out_hbm.at[cbase + k]
                return (
                    pltpu.make_async_copy(
                        rows_v.at[b, pl.ds(0, 72)], row.at[pl.ds(0, 72)],
                        ssems[b]),
                    pltpu.make_async_copy(
                        rows_v.at[b, pl.ds(72, seq - 72)],
                        row.at[pl.ds(72, seq - 72)], ssems[b]),
                )

            gather_desc(0, 0).start()

            @pl.loop(0, chunk, step=nbuf)
            def _body(n):
                for b in range(nbuf):
                    k = n + b
                    bn = (b + 1) % nbuf

                    @pl.when(k < chunk)
                    def _():
                        gather_desc(k, b).wait()
                        for dsc in store_desc(k, b):
                            dsc.start()

                    @pl.when(k >= 2)
                    def _():
                        for dsc in store_desc(k - 2, bn):
                            dsc.wait()

                    @pl.when(k + 1 < chunk)
                    def _():
                        gather_desc(k + 1, bn).start()

            last_n = nbuf * ((chunk - 1) // nbuf)
            drained = last_n + nbuf - 3
            for k in range(max(0, drained + 1), chunk):
                for dsc in store_desc(k, k % nbuf):
                    dsc.wait()

    return _gather_kernel(idx, table)
